# baseline (speedup 1.0000x reference)
import functools

import jax
import jax.numpy as jnp
from jax import lax
from jax.experimental import pallas as pl
from jax.experimental.pallas import tpu as pltpu
from jax.experimental.pallas import tpu_sc as plsc

D_MODEL = 1024
N_SEQ = 4
L_SEQ = 4096

_NC, _NS = 2, 16
_NW = _NC * _NS
_PER_W = (N_SEQ * L_SEQ) // _NW
_W_PER_SEQ = L_SEQ // _PER_W
_CHUNK = 16
_NCHUNK = _PER_W // _CHUNK


def _embed_body(table_hbm, idx_hbm, out_hbm, idx_v, rows0, rows1, rows2, rows3,
                gsem0, gsem1, gsem2, gsem3, ssem0, ssem1, ssem2, ssem3):
    wid = lax.axis_index("s") * _NC + lax.axis_index("c")
    seq = wid // _W_PER_SEQ
    col = (wid % _W_PER_SEQ) * _PER_W
    pltpu.sync_copy(idx_hbm.at[seq, pl.ds(col, _PER_W)], idx_v)

    bufs = (rows0, rows1, rows2, rows3)
    gsems = (gsem0, gsem1, gsem2, gsem3)
    ssems = (ssem0, ssem1, ssem2, ssem3)
    nbuf = 4

    def gather(c, b):
        return pltpu.async_copy(
            table_hbm.at[idx_v.at[pl.ds(c * _CHUNK, _CHUNK)]], bufs[b], gsems[b]
        )

    def scatter(c, b):
        return pltpu.async_copy(
            bufs[b], out_hbm.at[seq, pl.ds(col + c * _CHUNK, _CHUNK)], ssems[b]
        )

    for b in range(nbuf):
        gather(b, b)

    def ring(i, carry):
        for b in range(nbuf):
            c = nbuf * i + b
            pltpu.make_async_copy(
                table_hbm.at[pl.ds(0, _CHUNK)], bufs[b], gsems[b]
            ).wait()
            scatter(c, b)
            pltpu.make_async_copy(
                bufs[b], out_hbm.at[seq, pl.ds(col, _CHUNK)], ssems[b]
            ).wait()

            @pl.when(c + nbuf < _NCHUNK)
            def _():
                gather(c + nbuf, b)

        return carry

    lax.fori_loop(0, _NCHUNK // nbuf, ring, 0)


@jax.jit
def _embed_lookup(table, ids):
    mesh = plsc.VectorSubcoreMesh(core_axis_name="c", subcore_axis_name="s")
    run = pl.kernel(
        _embed_body,
        mesh=mesh,
        out_type=jax.ShapeDtypeStruct((N_SEQ, L_SEQ, D_MODEL), jnp.float32),
        scratch_types=[
            pltpu.VMEM((_PER_W,), jnp.int32),
            pltpu.VMEM((_CHUNK, D_MODEL), jnp.float32),
            pltpu.VMEM((_CHUNK, D_MODEL), jnp.float32),
            pltpu.VMEM((_CHUNK, D_MODEL), jnp.float32),
            pltpu.VMEM((_CHUNK, D_MODEL), jnp.float32),
            pltpu.SemaphoreType.DMA,
            pltpu.SemaphoreType.DMA,
            pltpu.SemaphoreType.DMA,
            pltpu.SemaphoreType.DMA,
            pltpu.SemaphoreType.DMA,
            pltpu.SemaphoreType.DMA,
            pltpu.SemaphoreType.DMA,
            pltpu.SemaphoreType.DMA,
        ],
    )
    return run(table, ids)


def kernel(input_ids, is_node, node_features, edge_index, mapping, embed_weight):
    return _embed_lookup(embed_weight, input_ids)
